# baseline (speedup 1.0000x reference)
import jax
import jax.numpy as jnp
from jax.experimental import pallas as pl
from jax.experimental.pallas import tpu as pltpu

N_NODES = 50000
N_ETYPES = 7
ET_DIM = 8
MEM_DIM = 16
T_DIM = 16
EV_DIM = 32
OUT_DIM = 32
HO = OUT_DIM


def _gru(x, h, w_ih, w_hh, b_ih, b_hh):
    gi = x @ w_ih.T + b_ih
    gh = h @ w_hh.T + b_hh
    ir, iz, inn = jnp.split(gi, 3, axis=-1)
    hr, hz, hn = jnp.split(gh, 3, axis=-1)
    r = jax.nn.sigmoid(ir + hr)
    z = jax.nn.sigmoid(iz + hz)
    nn_ = jnp.tanh(inn + r * hn)
    return (1.0 - z) * nn_ + z * h


def _tconv(x, src, dst, edge_attr, wq, bq, wk, bk, wv, bv, we, wskip, bskip):
    n = x.shape[0]
    q = (x @ wq.T + bq)[dst]
    kk = (x @ wk.T + bk)[src]
    vv = (x @ wv.T + bv)[src]
    e = edge_attr @ we.T
    kk = kk + e
    vv = vv + e
    logits = (q * kk).sum(-1) / jnp.sqrt(float(OUT_DIM))
    m = jax.ops.segment_max(logits, dst, num_segments=n)
    m = jnp.where(jnp.isfinite(m), m, 0.0)
    ex = jnp.exp(logits - m[dst])
    den = jax.ops.segment_sum(ex, dst, num_segments=n)
    alpha = ex / jnp.maximum(den[dst], 1e-16)
    out = jax.ops.segment_sum(vv * alpha[:, None], dst, num_segments=n)
    return out + x @ wskip.T + bskip


def _final_lin_kernel(h_ref, w_ref, b_ref, o_ref):
    o_ref[...] = h_ref[...] @ w_ref[...].T + b_ref[...]


def _final_linear(h, lin_w, lin_b):
    n = h.shape[0]
    blk = 400
    return pl.pallas_call(
        _final_lin_kernel,
        grid=(n // blk,),
        in_specs=[
            pl.BlockSpec((blk, HO), lambda i: (i, 0)),
            pl.BlockSpec((OUT_DIM, HO), lambda i: (0, 0)),
            pl.BlockSpec((1, OUT_DIM), lambda i: (0, 0)),
        ],
        out_specs=pl.BlockSpec((blk, OUT_DIM), lambda i: (i, 0)),
        out_shape=jax.ShapeDtypeStruct((n, OUT_DIM), jnp.float32),
    )(h, lin_w, lin_b.reshape(1, OUT_DIM))


def kernel(node_event_type_ids, node_event_node_ids, node_event_embeddings,
           node_event_timestamps, node_event_mask, edge_event_type_ids,
           edge_event_src_ids, edge_event_dst_ids, edge_event_edge_ids,
           edge_event_embeddings, edge_event_timestamps, edge_event_mask,
           memory, node_features, edge_index, edge_features, edge_timestamps,
           edge_last_update, type_emb, time_w, time_b, gru_w_ih, gru_w_hh,
           gru_b_ih, gru_b_hh, b1_wq, b1_bq, b1_wk, b1_bk, b1_wv, b1_bv,
           b1_we, b1_wskip, b1_bskip, b2_wq, b2_bq, b2_wk, b2_bk, b2_wv,
           b2_bv, b2_we, b2_wskip, b2_bskip, lin_w, lin_b):
    tw, tb = time_w, time_b
    mem = memory
    nte = type_emb[node_event_type_ids]
    nmem = mem[node_event_node_ids]
    nt = jnp.cos(node_event_timestamps[:, None] * tw + tb)
    node_msgs = jnp.concatenate(
        [nte, nmem, jnp.zeros_like(nmem), nt, node_event_embeddings],
        axis=-1) * node_event_mask[:, None]
    ete = type_emb[edge_event_type_ids]
    smem = mem[edge_event_src_ids]
    dmem = mem[edge_event_dst_ids]
    rel = edge_event_timestamps - edge_last_update[edge_event_edge_ids]
    rt = jnp.cos(rel[:, None] * tw + tb)
    src_msgs = jnp.concatenate(
        [ete, smem, dmem, rt, edge_event_embeddings],
        axis=-1) * edge_event_mask[:, None]
    dst_msgs = jnp.concatenate(
        [ete, dmem, smem, rt, edge_event_embeddings],
        axis=-1) * edge_event_mask[:, None]
    ev_ids = jnp.concatenate(
        [node_event_node_ids, edge_event_src_ids, edge_event_dst_ids])
    msgs = jnp.concatenate([node_msgs, src_msgs, dst_msgs], axis=0)
    agg = jax.ops.segment_sum(msgs, ev_ids, num_segments=N_NODES)
    has_ev = jnp.zeros((N_NODES,), dtype=bool).at[ev_ids].set(True)
    new_mem = _gru(agg, mem, gru_w_ih, gru_w_hh, gru_b_ih, gru_b_hh)
    mem2 = jnp.where(has_ev[:, None], new_mem, mem)
    rel_e = edge_timestamps - edge_last_update
    rel_emb = jnp.cos(rel_e[:, None] * tw + tb)
    x = jnp.concatenate([node_features, mem2], axis=-1)
    edge_attr = jnp.concatenate([rel_emb, edge_features], axis=-1)
    src = edge_index[0]
    dst = edge_index[1]
    h = _tconv(x, src, dst, edge_attr, b1_wq, b1_bq, b1_wk, b1_bk, b1_wv,
               b1_bv, b1_we, b1_wskip, b1_bskip)
    h = _tconv(jnp.concatenate([h, x], axis=-1), src, dst, edge_attr, b2_wq,
               b2_bq, b2_wk, b2_bk, b2_wv, b2_bv, b2_we, b2_wskip, b2_bskip)
    return _final_linear(h, lin_w, lin_b)


# speedup vs baseline: 4.0729x; 4.0729x over previous
import functools
import math

import jax
import jax.numpy as jnp
from jax import lax
from jax.experimental import pallas as pl
from jax.experimental.pallas import tpu as pltpu
from jax.experimental.pallas import tpu_sc as plsc

N_NODES = 50000
N_EDGES = 800000
MEM_DIM = 16
T_DIM = 16
EV_DIM = 32
OUT_DIM = 32
D = 32

NP = 50176
EP = 802816
NBLK = 512
EBLK = 4096
CHUNK = 128
CPW = EP // 32 // CHUNK
STRIPE = NP // 16
ZCH = 112
INV_SQRT_D = 1.0 / math.sqrt(float(D))


def _conv_sc_entry(src_hbm, dst_hbm, e_hbm, q_hbm, k_hbm, v_hbm,
                   out_hbm, den_hbm,
                   idx_s, idx_d, qr, kr, vr, er, val, sv, zb, zbd,
                   shared_out, shared_den, sem, semk, semv):
    c = lax.axis_index("c")
    s = lax.axis_index("s")
    wid = c * 16 + s
    z16 = jnp.zeros((16,), jnp.float32)

    def _zrow(i, _):
        zb[i, pl.ds(0, 16)] = z16
        zb[i, pl.ds(16, 16)] = z16
        return _
    lax.fori_loop(0, ZCH, _zrow, 0)
    for i in range(7):
        zbd[pl.ds(i * 16, 16)] = z16

    def _zsh(j, _):
        off = s * STRIPE + j * ZCH
        pltpu.sync_copy(zb, shared_out.at[pl.ds(off, ZCH)])
        pltpu.sync_copy(zbd, shared_den.at[pl.ds(off, ZCH)])
        return _
    lax.fori_loop(0, STRIPE // ZCH, _zsh, 0)
    plsc.subcore_barrier()

    rows0 = lax.iota(jnp.int32, 16)

    def _chunk(j, _):
        b = wid * (CPW * CHUNK) + j * CHUNK
        pltpu.sync_copy(src_hbm.at[pl.ds(b, CHUNK)], idx_s)
        pltpu.sync_copy(dst_hbm.at[pl.ds(b, CHUNK)], idx_d)
        dq = pltpu.async_copy(q_hbm.at[idx_d], qr, sem)
        dk = pltpu.async_copy(k_hbm.at[idx_s], kr, semk)
        dv = pltpu.async_copy(v_hbm.at[idx_s], vr, semv)
        pltpu.sync_copy(e_hbm.at[pl.ds(b, CHUNK)], er)
        dq.wait()
        dk.wait()
        dv.wait()

        def _group(g, _):
            rows = rows0 + g * 16
            acc = jnp.zeros((16,), jnp.float32)
            for d in range(D):
                col = jnp.full((16,), d, jnp.int32)
                qd = plsc.load_gather(qr, [rows, col])
                kd = plsc.load_gather(kr, [rows, col])
                ed = plsc.load_gather(er, [rows, col])
                acc = acc + qd * (kd + ed)
            sexp = jnp.exp(acc * INV_SQRT_D)
            sv[pl.ds(g * 16, 16)] = sexp
            for d in range(D):
                col = jnp.full((16,), d, jnp.int32)
                vd = plsc.load_gather(vr, [rows, col])
                ed = plsc.load_gather(er, [rows, col])
                plsc.store_scatter(val, [rows, col], (vd + ed) * sexp)
            return _
        lax.fori_loop(0, CHUNK // 16, _group, 0)

        pltpu.sync_copy(val, shared_out.at[idx_d], add=True)
        pltpu.sync_copy(sv, shared_den.at[idx_d], add=True)
        return _
    lax.fori_loop(0, CPW, _chunk, 0)
    plsc.subcore_barrier()

    def _flush(j, _):
        off = s * STRIPE + j * ZCH
        pltpu.sync_copy(shared_out.at[pl.ds(off, ZCH)], zb)
        pltpu.sync_copy(zb, out_hbm.at[c, pl.ds(off, ZCH)])
        pltpu.sync_copy(shared_den.at[pl.ds(off, ZCH)], zbd)
        pltpu.sync_copy(zbd, den_hbm.at[pl.ds(c * NP + off, ZCH)])
        return _
    lax.fori_loop(0, STRIPE // ZCH, _flush, 0)


def _conv_sc(src_pad, dst_pad, e_pad, q_tab, k_tab, v_tab):
    mesh = plsc.VectorSubcoreMesh(core_axis_name="c", subcore_axis_name="s")
    f = pl.kernel(
        _conv_sc_entry,
        out_type=[
            jax.ShapeDtypeStruct((2, NP, D), jnp.float32),
            jax.ShapeDtypeStruct((2 * NP,), jnp.float32),
        ],
        mesh=mesh,
        compiler_params=pltpu.CompilerParams(
            needs_layout_passes=False, use_tc_tiling_on_sc=False),
        scratch_types=[
            pltpu.VMEM((CHUNK,), jnp.int32),
            pltpu.VMEM((CHUNK,), jnp.int32),
            pltpu.VMEM((CHUNK, D), jnp.float32),
            pltpu.VMEM((CHUNK, D), jnp.float32),
            pltpu.VMEM((CHUNK, D), jnp.float32),
            pltpu.VMEM((CHUNK, D), jnp.float32),
            pltpu.VMEM((CHUNK, D), jnp.float32),
            pltpu.VMEM((CHUNK,), jnp.float32),
            pltpu.VMEM((ZCH, D), jnp.float32),
            pltpu.VMEM((ZCH,), jnp.float32),
            pltpu.VMEM_SHARED((NP, D), jnp.float32),
            pltpu.VMEM_SHARED((NP,), jnp.float32),
            pltpu.SemaphoreType.DMA,
            pltpu.SemaphoreType.DMA,
            pltpu.SemaphoreType.DMA,
        ],
    )
    return f(src_pad, dst_pad, e_pad, q_tab, k_tab, v_tab)


def _eprep_kernel(ets_ref, elu_ref, ef_ref, tw_ref, tb_ref, w1_ref, w2_ref,
                  e1_ref, e2_ref):
    rel = ets_ref[...] - elu_ref[...]
    re = jnp.cos(rel * tw_ref[...] + tb_ref[...])
    ea = jnp.concatenate([re, ef_ref[...]], axis=1)
    e1_ref[...] = ea @ w1_ref[...].T
    e2_ref[...] = ea @ w2_ref[...].T


def _eprep(ets_p, elu_p, ef_p, tw, tb, w1, w2):
    grid = (EP // EBLK,)
    return pl.pallas_call(
        _eprep_kernel,
        grid=grid,
        in_specs=[
            pl.BlockSpec((EBLK, 1), lambda i: (i, 0)),
            pl.BlockSpec((EBLK, 1), lambda i: (i, 0)),
            pl.BlockSpec((EBLK, EV_DIM), lambda i: (i, 0)),
            pl.BlockSpec((1, T_DIM), lambda i: (0, 0)),
            pl.BlockSpec((1, T_DIM), lambda i: (0, 0)),
            pl.BlockSpec((D, T_DIM + EV_DIM), lambda i: (0, 0)),
            pl.BlockSpec((D, T_DIM + EV_DIM), lambda i: (0, 0)),
        ],
        out_specs=[
            pl.BlockSpec((EBLK, D), lambda i: (i, 0)),
            pl.BlockSpec((EBLK, D), lambda i: (i, 0)),
        ],
        out_shape=[
            jax.ShapeDtypeStruct((EP, D), jnp.float32),
            jax.ShapeDtypeStruct((EP, D), jnp.float32),
        ],
    )(ets_p, elu_p, ef_p, tw, tb, w1, w2)


def _gru_proj_kernel(agg_ref, mem_ref, hv_ref, nf_ref,
                     wih_ref, whh_ref, bih_ref, bhh_ref,
                     wq_ref, bq_ref, wk_ref, bk_ref, wv_ref, bv_ref,
                     ws_ref, bs_ref,
                     x_ref, q_ref, k_ref, v_ref, sk_ref):
    mem = mem_ref[...]
    gi = agg_ref[...] @ wih_ref[...].T + bih_ref[...]
    gh = mem @ whh_ref[...].T + bhh_ref[...]
    r = jax.nn.sigmoid(gi[:, 0:16] + gh[:, 0:16])
    z = jax.nn.sigmoid(gi[:, 16:32] + gh[:, 16:32])
    n = jnp.tanh(gi[:, 32:48] + r * gh[:, 32:48])
    newm = (1.0 - z) * n + z * mem
    mem2 = jnp.where(hv_ref[...] > 0, newm, mem)
    x = jnp.concatenate([nf_ref[...], mem2], axis=1)
    x_ref[...] = x
    q_ref[...] = x @ wq_ref[...].T + bq_ref[...]
    k_ref[...] = x @ wk_ref[...].T + bk_ref[...]
    v_ref[...] = x @ wv_ref[...].T + bv_ref[...]
    sk_ref[...] = x @ ws_ref[...].T + bs_ref[...]


def _gru_proj(agg_p, mem_p, hv_p, nf_p, wih, whh, bih, bhh,
              wq, bq, wk, bk, wv, bv, ws, bs):
    grid = (NP // NBLK,)
    GIN = EV_DIM + MEM_DIM
    row = lambda i: (i, 0)
    fixed = lambda i: (0, 0)
    return pl.pallas_call(
        _gru_proj_kernel,
        grid=grid,
        in_specs=[
            pl.BlockSpec((NBLK, 88), row),
            pl.BlockSpec((NBLK, MEM_DIM), row),
            pl.BlockSpec((NBLK, 1), row),
            pl.BlockSpec((NBLK, EV_DIM), row),
            pl.BlockSpec((48, 88), fixed),
            pl.BlockSpec((48, MEM_DIM), fixed),
            pl.BlockSpec((1, 48), fixed),
            pl.BlockSpec((1, 48), fixed),
            pl.BlockSpec((D, GIN), fixed),
            pl.BlockSpec((1, D), fixed),
            pl.BlockSpec((D, GIN), fixed),
            pl.BlockSpec((1, D), fixed),
            pl.BlockSpec((D, GIN), fixed),
            pl.BlockSpec((1, D), fixed),
            pl.BlockSpec((D, GIN), fixed),
            pl.BlockSpec((1, D), fixed),
        ],
        out_specs=[
            pl.BlockSpec((NBLK, GIN), row),
            pl.BlockSpec((NBLK, D), row),
            pl.BlockSpec((NBLK, D), row),
            pl.BlockSpec((NBLK, D), row),
            pl.BlockSpec((NBLK, D), row),
        ],
        out_shape=[
            jax.ShapeDtypeStruct((NP, GIN), jnp.float32),
            jax.ShapeDtypeStruct((NP, D), jnp.float32),
            jax.ShapeDtypeStruct((NP, D), jnp.float32),
            jax.ShapeDtypeStruct((NP, D), jnp.float32),
            jax.ShapeDtypeStruct((NP, D), jnp.float32),
        ],
    )(agg_p, mem_p, hv_p, nf_p, wih, whh, bih, bhh,
      wq, bq, wk, bk, wv, bv, ws, bs)


def _norm_proj2_kernel(o0_ref, o1_ref, d0_ref, d1_ref, sk_ref, x_ref,
                       wq_ref, bq_ref, wk_ref, bk_ref, wv_ref, bv_ref,
                       ws_ref, bs_ref,
                       q_ref, k_ref, v_ref, sk2_ref):
    den = jnp.maximum(d0_ref[...] + d1_ref[...], 1e-16)
    h1 = (o0_ref[...] + o1_ref[...]) / den + sk_ref[...]
    x2 = jnp.concatenate([h1, x_ref[...]], axis=1)
    q_ref[...] = x2 @ wq_ref[...].T + bq_ref[...]
    k_ref[...] = x2 @ wk_ref[...].T + bk_ref[...]
    v_ref[...] = x2 @ wv_ref[...].T + bv_ref[...]
    sk2_ref[...] = x2 @ ws_ref[...].T + bs_ref[...]


def _norm_proj2(o0, o1, d0, d1, sk, x, wq, bq, wk, bk, wv, bv, ws, bs):
    grid = (NP // NBLK,)
    D2 = D + EV_DIM + MEM_DIM
    row = lambda i: (i, 0)
    fixed = lambda i: (0, 0)
    return pl.pallas_call(
        _norm_proj2_kernel,
        grid=grid,
        in_specs=[
            pl.BlockSpec((NBLK, D), row),
            pl.BlockSpec((NBLK, D), row),
            pl.BlockSpec((NBLK, 1), row),
            pl.BlockSpec((NBLK, 1), row),
            pl.BlockSpec((NBLK, D), row),
            pl.BlockSpec((NBLK, EV_DIM + MEM_DIM), row),
            pl.BlockSpec((D, D2), fixed),
            pl.BlockSpec((1, D), fixed),
            pl.BlockSpec((D, D2), fixed),
            pl.BlockSpec((1, D), fixed),
            pl.BlockSpec((D, D2), fixed),
            pl.BlockSpec((1, D), fixed),
            pl.BlockSpec((D, D2), fixed),
            pl.BlockSpec((1, D), fixed),
        ],
        out_specs=[
            pl.BlockSpec((NBLK, D), row),
            pl.BlockSpec((NBLK, D), row),
            pl.BlockSpec((NBLK, D), row),
            pl.BlockSpec((NBLK, D), row),
        ],
        out_shape=[
            jax.ShapeDtypeStruct((NP, D), jnp.float32),
            jax.ShapeDtypeStruct((NP, D), jnp.float32),
            jax.ShapeDtypeStruct((NP, D), jnp.float32),
            jax.ShapeDtypeStruct((NP, D), jnp.float32),
        ],
    )(o0, o1, d0, d1, sk, x, wq, bq, wk, bk, wv, bv, ws, bs)


def _norm_final_kernel(o0_ref, o1_ref, d0_ref, d1_ref, sk_ref,
                       w_ref, b_ref, out_ref):
    den = jnp.maximum(d0_ref[...] + d1_ref[...], 1e-16)
    h2 = (o0_ref[...] + o1_ref[...]) / den + sk_ref[...]
    out_ref[...] = h2 @ w_ref[...].T + b_ref[...]


def _norm_final(o0, o1, d0, d1, sk, w, b):
    grid = (NP // NBLK,)
    row = lambda i: (i, 0)
    fixed = lambda i: (0, 0)
    return pl.pallas_call(
        _norm_final_kernel,
        grid=grid,
        in_specs=[
            pl.BlockSpec((NBLK, D), row),
            pl.BlockSpec((NBLK, D), row),
            pl.BlockSpec((NBLK, 1), row),
            pl.BlockSpec((NBLK, 1), row),
            pl.BlockSpec((NBLK, D), row),
            pl.BlockSpec((OUT_DIM, D), fixed),
            pl.BlockSpec((1, OUT_DIM), fixed),
        ],
        out_specs=pl.BlockSpec((NBLK, OUT_DIM), row),
        out_shape=jax.ShapeDtypeStruct((NP, OUT_DIM), jnp.float32),
    )(o0, o1, d0, d1, sk, w, b)


def kernel(node_event_type_ids, node_event_node_ids, node_event_embeddings,
           node_event_timestamps, node_event_mask, edge_event_type_ids,
           edge_event_src_ids, edge_event_dst_ids, edge_event_edge_ids,
           edge_event_embeddings, edge_event_timestamps, edge_event_mask,
           memory, node_features, edge_index, edge_features, edge_timestamps,
           edge_last_update, type_emb, time_w, time_b, gru_w_ih, gru_w_hh,
           gru_b_ih, gru_b_hh, b1_wq, b1_bq, b1_wk, b1_bk, b1_wv, b1_bv,
           b1_we, b1_wskip, b1_bskip, b2_wq, b2_bq, b2_wk, b2_bk, b2_wv,
           b2_bv, b2_we, b2_wskip, b2_bskip, lin_w, lin_b):
    tw, tb = time_w, time_b
    mem = memory

    nte = type_emb[node_event_type_ids]
    nmem = mem[node_event_node_ids]
    nt = jnp.cos(node_event_timestamps[:, None] * tw + tb)
    node_msgs = jnp.concatenate(
        [nte, nmem, jnp.zeros_like(nmem), nt, node_event_embeddings],
        axis=-1) * node_event_mask[:, None]
    ete = type_emb[edge_event_type_ids]
    smem = mem[edge_event_src_ids]
    dmem = mem[edge_event_dst_ids]
    rel = edge_event_timestamps - edge_last_update[edge_event_edge_ids]
    rt = jnp.cos(rel[:, None] * tw + tb)
    src_msgs = jnp.concatenate(
        [ete, smem, dmem, rt, edge_event_embeddings],
        axis=-1) * edge_event_mask[:, None]
    dst_msgs = jnp.concatenate(
        [ete, dmem, smem, rt, edge_event_embeddings],
        axis=-1) * edge_event_mask[:, None]
    ev_ids = jnp.concatenate(
        [node_event_node_ids, edge_event_src_ids, edge_event_dst_ids])
    msgs = jnp.concatenate([node_msgs, src_msgs, dst_msgs], axis=0)
    agg = jax.ops.segment_sum(msgs, ev_ids, num_segments=N_NODES)
    has_ev = jnp.zeros((N_NODES,), dtype=jnp.float32).at[ev_ids].set(1.0)

    padn = ((0, NP - N_NODES), (0, 0))
    agg_p = jnp.pad(agg, padn)
    mem_p = jnp.pad(mem, padn)
    hv_p = jnp.pad(has_ev[:, None], padn)
    nf_p = jnp.pad(node_features, padn)

    pade = (0, EP - N_EDGES)
    ets_p = jnp.pad(edge_timestamps, pade)[:, None]
    elu_p = jnp.pad(edge_last_update, pade)[:, None]
    ef_p = jnp.pad(edge_features, (pade, (0, 0)))
    src_p = jnp.pad(edge_index[0].astype(jnp.int32), pade,
                    constant_values=N_NODES)
    dst_p = jnp.pad(edge_index[1].astype(jnp.int32), pade,
                    constant_values=N_NODES)

    e1_p, e2_p = _eprep(ets_p, elu_p, ef_p, tw[None, :], tb[None, :],
                        b1_we, b2_we)

    x_p, q1, k1, v1, sk1 = _gru_proj(
        agg_p, mem_p, hv_p, nf_p,
        gru_w_ih, gru_w_hh, gru_b_ih[None, :], gru_b_hh[None, :],
        b1_wq, b1_bq[None, :], b1_wk, b1_bk[None, :],
        b1_wv, b1_bv[None, :], b1_wskip, b1_bskip[None, :])

    out1, den1 = _conv_sc(src_p, dst_p, e1_p, q1, k1, v1)

    q2, k2, v2, sk2 = _norm_proj2(
        out1[0], out1[1], den1[:NP][:, None], den1[NP:][:, None], sk1, x_p,
        b2_wq, b2_bq[None, :], b2_wk, b2_bk[None, :],
        b2_wv, b2_bv[None, :], b2_wskip, b2_bskip[None, :])

    out2, den2 = _conv_sc(src_p, dst_p, e2_p, q2, k2, v2)

    res = _norm_final(out2[0], out2[1], den2[:NP][:, None], den2[NP:][:, None],
                      sk2, lin_w, lin_b[None, :])
    return res[:N_NODES]


# speedup vs baseline: 4.1177x; 1.0110x over previous
import functools
import math

import jax
import jax.numpy as jnp
from jax import lax
from jax.experimental import pallas as pl
from jax.experimental.pallas import tpu as pltpu
from jax.experimental.pallas import tpu_sc as plsc

N_NODES = 50000
N_EDGES = 800000
MEM_DIM = 16
T_DIM = 16
EV_DIM = 32
OUT_DIM = 32
D = 32

NP = 50176
EP = 802816
NBLK = 512
EBLK = 4096
CHUNK = 64
CPW = EP // 32 // CHUNK
STRIPE = NP // 16
ZCH = 112
INV_SQRT_D = 1.0 / math.sqrt(float(D))


def _conv_sc_entry(src_hbm, dst_hbm, e_hbm, q_hbm, k_hbm, v_hbm,
                   out_hbm, den_hbm,
                   idx_s, idx_d, qr, kr, vr, er, val, sv, zb, zbd,
                   shared_out, shared_den, sem, semk, semv, seme):
    c = lax.axis_index("c")
    s = lax.axis_index("s")
    wid = c * 16 + s
    z16 = jnp.zeros((16,), jnp.float32)

    def _zrow(i, _):
        zb[i, pl.ds(0, 16)] = z16
        zb[i, pl.ds(16, 16)] = z16
        return _
    lax.fori_loop(0, ZCH, _zrow, 0)
    for i in range(7):
        zbd[pl.ds(i * 16, 16)] = z16

    def _zsh(j, _):
        off = s * STRIPE + j * ZCH
        pltpu.sync_copy(zb, shared_out.at[pl.ds(off, ZCH)])
        pltpu.sync_copy(zbd, shared_den.at[pl.ds(off, ZCH)])
        return _
    lax.fori_loop(0, STRIPE // ZCH, _zsh, 0)
    plsc.subcore_barrier()

    rows0 = lax.iota(jnp.int32, 16)
    base0 = wid * (CPW * CHUNK)

    def _prefetch(j, b):
        bn = base0 + j * CHUNK
        pltpu.sync_copy(src_hbm.at[pl.ds(bn, CHUNK)], idx_s.at[b])
        pltpu.sync_copy(dst_hbm.at[pl.ds(bn, CHUNK)], idx_d.at[b])
        pltpu.async_copy(q_hbm.at[idx_d.at[b]], qr.at[b], sem)
        pltpu.async_copy(k_hbm.at[idx_s.at[b]], kr.at[b], semk)
        pltpu.async_copy(v_hbm.at[idx_s.at[b]], vr.at[b], semv)
        pltpu.async_copy(e_hbm.at[pl.ds(bn, CHUNK)], er.at[b], seme)

    _prefetch(0, 0)

    def _chunk(j, carry):
        b = lax.rem(j, 2)
        nb = 1 - b

        @pl.when(j + 1 < CPW)
        def _pf():
            _prefetch(j + 1, nb)

        pltpu.make_async_copy(q_hbm.at[idx_d.at[b]], qr.at[b], sem).wait()
        pltpu.make_async_copy(k_hbm.at[idx_s.at[b]], kr.at[b], semk).wait()
        pltpu.make_async_copy(v_hbm.at[idx_s.at[b]], vr.at[b], semv).wait()
        pltpu.make_async_copy(
            e_hbm.at[pl.ds(base0, CHUNK)], er.at[b], seme).wait()
        qb, kb, vb, eb = qr.at[b], kr.at[b], vr.at[b], er.at[b]

        def _group(g, _):
            rows = rows0 + g * 16
            acc = jnp.zeros((16,), jnp.float32)
            for d in range(D):
                col = jnp.full((16,), d, jnp.int32)
                qd = plsc.load_gather(qb, [rows, col])
                kd = plsc.load_gather(kb, [rows, col])
                ed = plsc.load_gather(eb, [rows, col])
                acc = acc + qd * (kd + ed)
            sexp = jnp.exp(acc * INV_SQRT_D)
            sv[pl.ds(g * 16, 16)] = sexp
            for d in range(D):
                col = jnp.full((16,), d, jnp.int32)
                vd = plsc.load_gather(vb, [rows, col])
                ed = plsc.load_gather(eb, [rows, col])
                plsc.store_scatter(val, [rows, col], (vd + ed) * sexp)
            return _
        lax.fori_loop(0, CHUNK // 16, _group, 0)

        pltpu.sync_copy(val, shared_out.at[idx_d.at[b]], add=True)
        pltpu.sync_copy(sv, shared_den.at[idx_d.at[b]], add=True)
        return carry
    lax.fori_loop(0, CPW, _chunk, 0)
    plsc.subcore_barrier()

    def _flush(j, _):
        off = s * STRIPE + j * ZCH
        pltpu.sync_copy(shared_out.at[pl.ds(off, ZCH)], zb)
        pltpu.sync_copy(zb, out_hbm.at[c, pl.ds(off, ZCH)])
        pltpu.sync_copy(shared_den.at[pl.ds(off, ZCH)], zbd)
        pltpu.sync_copy(zbd, den_hbm.at[pl.ds(c * NP + off, ZCH)])
        return _
    lax.fori_loop(0, STRIPE // ZCH, _flush, 0)


def _conv_sc(src_pad, dst_pad, e_pad, q_tab, k_tab, v_tab):
    mesh = plsc.VectorSubcoreMesh(core_axis_name="c", subcore_axis_name="s")
    f = pl.kernel(
        _conv_sc_entry,
        out_type=[
            jax.ShapeDtypeStruct((2, NP, D), jnp.float32),
            jax.ShapeDtypeStruct((2 * NP,), jnp.float32),
        ],
        mesh=mesh,
        compiler_params=pltpu.CompilerParams(
            needs_layout_passes=False, use_tc_tiling_on_sc=False),
        scratch_types=[
            pltpu.VMEM((2, CHUNK), jnp.int32),
            pltpu.VMEM((2, CHUNK), jnp.int32),
            pltpu.VMEM((2, CHUNK, D), jnp.float32),
            pltpu.VMEM((2, CHUNK, D), jnp.float32),
            pltpu.VMEM((2, CHUNK, D), jnp.float32),
            pltpu.VMEM((2, CHUNK, D), jnp.float32),
            pltpu.VMEM((CHUNK, D), jnp.float32),
            pltpu.VMEM((CHUNK,), jnp.float32),
            pltpu.VMEM((ZCH, D), jnp.float32),
            pltpu.VMEM((ZCH,), jnp.float32),
            pltpu.VMEM_SHARED((NP, D), jnp.float32),
            pltpu.VMEM_SHARED((NP,), jnp.float32),
            pltpu.SemaphoreType.DMA,
            pltpu.SemaphoreType.DMA,
            pltpu.SemaphoreType.DMA,
            pltpu.SemaphoreType.DMA,
        ],
    )
    return f(src_pad, dst_pad, e_pad, q_tab, k_tab, v_tab)


def _eprep_kernel(ets_ref, elu_ref, ef_ref, tw_ref, tb_ref, w1_ref, w2_ref,
                  e1_ref, e2_ref):
    rel = ets_ref[...] - elu_ref[...]
    re = jnp.cos(rel * tw_ref[...] + tb_ref[...])
    ea = jnp.concatenate([re, ef_ref[...]], axis=1)
    e1_ref[...] = ea @ w1_ref[...].T
    e2_ref[...] = ea @ w2_ref[...].T


def _eprep(ets_p, elu_p, ef_p, tw, tb, w1, w2):
    grid = (EP // EBLK,)
    return pl.pallas_call(
        _eprep_kernel,
        grid=grid,
        in_specs=[
            pl.BlockSpec((EBLK, 1), lambda i: (i, 0)),
            pl.BlockSpec((EBLK, 1), lambda i: (i, 0)),
            pl.BlockSpec((EBLK, EV_DIM), lambda i: (i, 0)),
            pl.BlockSpec((1, T_DIM), lambda i: (0, 0)),
            pl.BlockSpec((1, T_DIM), lambda i: (0, 0)),
            pl.BlockSpec((D, T_DIM + EV_DIM), lambda i: (0, 0)),
            pl.BlockSpec((D, T_DIM + EV_DIM), lambda i: (0, 0)),
        ],
        out_specs=[
            pl.BlockSpec((EBLK, D), lambda i: (i, 0)),
            pl.BlockSpec((EBLK, D), lambda i: (i, 0)),
        ],
        out_shape=[
            jax.ShapeDtypeStruct((EP, D), jnp.float32),
            jax.ShapeDtypeStruct((EP, D), jnp.float32),
        ],
    )(ets_p, elu_p, ef_p, tw, tb, w1, w2)


def _gru_proj_kernel(agg_ref, mem_ref, hv_ref, nf_ref,
                     wih_ref, whh_ref, bih_ref, bhh_ref,
                     wq_ref, bq_ref, wk_ref, bk_ref, wv_ref, bv_ref,
                     ws_ref, bs_ref,
                     x_ref, q_ref, k_ref, v_ref, sk_ref):
    mem = mem_ref[...]
    gi = agg_ref[...] @ wih_ref[...].T + bih_ref[...]
    gh = mem @ whh_ref[...].T + bhh_ref[...]
    r = jax.nn.sigmoid(gi[:, 0:16] + gh[:, 0:16])
    z = jax.nn.sigmoid(gi[:, 16:32] + gh[:, 16:32])
    n = jnp.tanh(gi[:, 32:48] + r * gh[:, 32:48])
    newm = (1.0 - z) * n + z * mem
    mem2 = jnp.where(hv_ref[...] > 0, newm, mem)
    x = jnp.concatenate([nf_ref[...], mem2], axis=1)
    x_ref[...] = x
    q_ref[...] = x @ wq_ref[...].T + bq_ref[...]
    k_ref[...] = x @ wk_ref[...].T + bk_ref[...]
    v_ref[...] = x @ wv_ref[...].T + bv_ref[...]
    sk_ref[...] = x @ ws_ref[...].T + bs_ref[...]


def _gru_proj(agg_p, mem_p, hv_p, nf_p, wih, whh, bih, bhh,
              wq, bq, wk, bk, wv, bv, ws, bs):
    grid = (NP // NBLK,)
    GIN = EV_DIM + MEM_DIM
    row = lambda i: (i, 0)
    fixed = lambda i: (0, 0)
    return pl.pallas_call(
        _gru_proj_kernel,
        grid=grid,
        in_specs=[
            pl.BlockSpec((NBLK, 88), row),
            pl.BlockSpec((NBLK, MEM_DIM), row),
            pl.BlockSpec((NBLK, 1), row),
            pl.BlockSpec((NBLK, EV_DIM), row),
            pl.BlockSpec((48, 88), fixed),
            pl.BlockSpec((48, MEM_DIM), fixed),
            pl.BlockSpec((1, 48), fixed),
            pl.BlockSpec((1, 48), fixed),
            pl.BlockSpec((D, GIN), fixed),
            pl.BlockSpec((1, D), fixed),
            pl.BlockSpec((D, GIN), fixed),
            pl.BlockSpec((1, D), fixed),
            pl.BlockSpec((D, GIN), fixed),
            pl.BlockSpec((1, D), fixed),
            pl.BlockSpec((D, GIN), fixed),
            pl.BlockSpec((1, D), fixed),
        ],
        out_specs=[
            pl.BlockSpec((NBLK, GIN), row),
            pl.BlockSpec((NBLK, D), row),
            pl.BlockSpec((NBLK, D), row),
            pl.BlockSpec((NBLK, D), row),
            pl.BlockSpec((NBLK, D), row),
        ],
        out_shape=[
            jax.ShapeDtypeStruct((NP, GIN), jnp.float32),
            jax.ShapeDtypeStruct((NP, D), jnp.float32),
            jax.ShapeDtypeStruct((NP, D), jnp.float32),
            jax.ShapeDtypeStruct((NP, D), jnp.float32),
            jax.ShapeDtypeStruct((NP, D), jnp.float32),
        ],
    )(agg_p, mem_p, hv_p, nf_p, wih, whh, bih, bhh,
      wq, bq, wk, bk, wv, bv, ws, bs)


def _norm_proj2_kernel(o0_ref, o1_ref, d0_ref, d1_ref, sk_ref, x_ref,
                       wq_ref, bq_ref, wk_ref, bk_ref, wv_ref, bv_ref,
                       ws_ref, bs_ref,
                       q_ref, k_ref, v_ref, sk2_ref):
    den = jnp.maximum(d0_ref[...] + d1_ref[...], 1e-16)
    h1 = (o0_ref[...] + o1_ref[...]) / den + sk_ref[...]
    x2 = jnp.concatenate([h1, x_ref[...]], axis=1)
    q_ref[...] = x2 @ wq_ref[...].T + bq_ref[...]
    k_ref[...] = x2 @ wk_ref[...].T + bk_ref[...]
    v_ref[...] = x2 @ wv_ref[...].T + bv_ref[...]
    sk2_ref[...] = x2 @ ws_ref[...].T + bs_ref[...]


def _norm_proj2(o0, o1, d0, d1, sk, x, wq, bq, wk, bk, wv, bv, ws, bs):
    grid = (NP // NBLK,)
    D2 = D + EV_DIM + MEM_DIM
    row = lambda i: (i, 0)
    fixed = lambda i: (0, 0)
    return pl.pallas_call(
        _norm_proj2_kernel,
        grid=grid,
        in_specs=[
            pl.BlockSpec((NBLK, D), row),
            pl.BlockSpec((NBLK, D), row),
            pl.BlockSpec((NBLK, 1), row),
            pl.BlockSpec((NBLK, 1), row),
            pl.BlockSpec((NBLK, D), row),
            pl.BlockSpec((NBLK, EV_DIM + MEM_DIM), row),
            pl.BlockSpec((D, D2), fixed),
            pl.BlockSpec((1, D), fixed),
            pl.BlockSpec((D, D2), fixed),
            pl.BlockSpec((1, D), fixed),
            pl.BlockSpec((D, D2), fixed),
            pl.BlockSpec((1, D), fixed),
            pl.BlockSpec((D, D2), fixed),
            pl.BlockSpec((1, D), fixed),
        ],
        out_specs=[
            pl.BlockSpec((NBLK, D), row),
            pl.BlockSpec((NBLK, D), row),
            pl.BlockSpec((NBLK, D), row),
            pl.BlockSpec((NBLK, D), row),
        ],
        out_shape=[
            jax.ShapeDtypeStruct((NP, D), jnp.float32),
            jax.ShapeDtypeStruct((NP, D), jnp.float32),
            jax.ShapeDtypeStruct((NP, D), jnp.float32),
            jax.ShapeDtypeStruct((NP, D), jnp.float32),
        ],
    )(o0, o1, d0, d1, sk, x, wq, bq, wk, bk, wv, bv, ws, bs)


def _norm_final_kernel(o0_ref, o1_ref, d0_ref, d1_ref, sk_ref,
                       w_ref, b_ref, out_ref):
    den = jnp.maximum(d0_ref[...] + d1_ref[...], 1e-16)
    h2 = (o0_ref[...] + o1_ref[...]) / den + sk_ref[...]
    out_ref[...] = h2 @ w_ref[...].T + b_ref[...]


def _norm_final(o0, o1, d0, d1, sk, w, b):
    grid = (NP // NBLK,)
    row = lambda i: (i, 0)
    fixed = lambda i: (0, 0)
    return pl.pallas_call(
        _norm_final_kernel,
        grid=grid,
        in_specs=[
            pl.BlockSpec((NBLK, D), row),
            pl.BlockSpec((NBLK, D), row),
            pl.BlockSpec((NBLK, 1), row),
            pl.BlockSpec((NBLK, 1), row),
            pl.BlockSpec((NBLK, D), row),
            pl.BlockSpec((OUT_DIM, D), fixed),
            pl.BlockSpec((1, OUT_DIM), fixed),
        ],
        out_specs=pl.BlockSpec((NBLK, OUT_DIM), row),
        out_shape=jax.ShapeDtypeStruct((NP, OUT_DIM), jnp.float32),
    )(o0, o1, d0, d1, sk, w, b)


def kernel(node_event_type_ids, node_event_node_ids, node_event_embeddings,
           node_event_timestamps, node_event_mask, edge_event_type_ids,
           edge_event_src_ids, edge_event_dst_ids, edge_event_edge_ids,
           edge_event_embeddings, edge_event_timestamps, edge_event_mask,
           memory, node_features, edge_index, edge_features, edge_timestamps,
           edge_last_update, type_emb, time_w, time_b, gru_w_ih, gru_w_hh,
           gru_b_ih, gru_b_hh, b1_wq, b1_bq, b1_wk, b1_bk, b1_wv, b1_bv,
           b1_we, b1_wskip, b1_bskip, b2_wq, b2_bq, b2_wk, b2_bk, b2_wv,
           b2_bv, b2_we, b2_wskip, b2_bskip, lin_w, lin_b):
    tw, tb = time_w, time_b
    mem = memory

    nte = type_emb[node_event_type_ids]
    nmem = mem[node_event_node_ids]
    nt = jnp.cos(node_event_timestamps[:, None] * tw + tb)
    node_msgs = jnp.concatenate(
        [nte, nmem, jnp.zeros_like(nmem), nt, node_event_embeddings],
        axis=-1) * node_event_mask[:, None]
    ete = type_emb[edge_event_type_ids]
    smem = mem[edge_event_src_ids]
    dmem = mem[edge_event_dst_ids]
    rel = edge_event_timestamps - edge_last_update[edge_event_edge_ids]
    rt = jnp.cos(rel[:, None] * tw + tb)
    src_msgs = jnp.concatenate(
        [ete, smem, dmem, rt, edge_event_embeddings],
        axis=-1) * edge_event_mask[:, None]
    dst_msgs = jnp.concatenate(
        [ete, dmem, smem, rt, edge_event_embeddings],
        axis=-1) * edge_event_mask[:, None]
    ev_ids = jnp.concatenate(
        [node_event_node_ids, edge_event_src_ids, edge_event_dst_ids])
    msgs = jnp.concatenate([node_msgs, src_msgs, dst_msgs], axis=0)
    agg = jax.ops.segment_sum(msgs, ev_ids, num_segments=N_NODES)
    has_ev = jnp.zeros((N_NODES,), dtype=jnp.float32).at[ev_ids].set(1.0)

    padn = ((0, NP - N_NODES), (0, 0))
    agg_p = jnp.pad(agg, padn)
    mem_p = jnp.pad(mem, padn)
    hv_p = jnp.pad(has_ev[:, None], padn)
    nf_p = jnp.pad(node_features, padn)

    pade = (0, EP - N_EDGES)
    ets_p = jnp.pad(edge_timestamps, pade)[:, None]
    elu_p = jnp.pad(edge_last_update, pade)[:, None]
    ef_p = jnp.pad(edge_features, (pade, (0, 0)))
    src_p = jnp.pad(edge_index[0].astype(jnp.int32), pade,
                    constant_values=N_NODES)
    dst_p = jnp.pad(edge_index[1].astype(jnp.int32), pade,
                    constant_values=N_NODES)

    e1_p, e2_p = _eprep(ets_p, elu_p, ef_p, tw[None, :], tb[None, :],
                        b1_we, b2_we)

    x_p, q1, k1, v1, sk1 = _gru_proj(
        agg_p, mem_p, hv_p, nf_p,
        gru_w_ih, gru_w_hh, gru_b_ih[None, :], gru_b_hh[None, :],
        b1_wq, b1_bq[None, :], b1_wk, b1_bk[None, :],
        b1_wv, b1_bv[None, :], b1_wskip, b1_bskip[None, :])

    out1, den1 = _conv_sc(src_p, dst_p, e1_p, q1, k1, v1)

    q2, k2, v2, sk2 = _norm_proj2(
        out1[0], out1[1], den1[:NP][:, None], den1[NP:][:, None], sk1, x_p,
        b2_wq, b2_bq[None, :], b2_wk, b2_bk[None, :],
        b2_wv, b2_bv[None, :], b2_wskip, b2_bskip[None, :])

    out2, den2 = _conv_sc(src_p, dst_p, e2_p, q2, k2, v2)

    res = _norm_final(out2[0], out2[1], den2[:NP][:, None], den2[NP:][:, None],
                      sk2, lin_w, lin_b[None, :])
    return res[:N_NODES]
